# baseline (speedup 1.0000x reference)
import jax
import jax.numpy as jnp
from jax.experimental import pallas as pl

_K = 40


def _knn_idx(xs, k):
    inner = -2.0 * jnp.einsum('bcn,bcm->bnm', xs, xs)
    sq = jnp.sum(xs * xs, axis=1)
    dist = inner + sq[:, :, None] + sq[:, None, :]
    _, idx = jax.lax.top_k(-dist, k)
    return idx


def _edge_conv(x, W, b, k, gamma=None, beta=None):
    xs = x[..., 0]
    B, C, N = xs.shape
    idx = _knn_idx(xs, k)
    xt = jnp.transpose(xs, (0, 2, 1))
    xj = xt[jnp.arange(B)[:, None, None], idx]
    xj = jnp.transpose(xj, (0, 3, 1, 2))
    xi = jnp.broadcast_to(xs[:, :, :, None], (B, C, N, k))
    feat = jnp.concatenate([xi, xj - xi], axis=1)
    out = jnp.einsum('oc,bcnk->bonk', W, feat) + b[None, :, None, None]
    if gamma is not None:
        mean = jnp.mean(out, axis=(0, 2, 3), keepdims=True)
        var = jnp.var(out, axis=(0, 2, 3), keepdims=True)
        out = (out - mean) / jnp.sqrt(var + 1e-5) * gamma[None, :, None, None] + beta[None, :, None, None]
    out = jax.nn.selu(out)
    return jnp.max(out, axis=3, keepdims=True)


_SELU_ALPHA = 1.6732632423543772
_SELU_SCALE = 1.0507009873554805


def _selu(x):
    return _SELU_SCALE * jnp.where(x > 0, x, _SELU_ALPHA * (jnp.exp(x) - 1.0))


def _final_kernel(feat_ref, w_ref, b_ref, o_ref):
    acc = jnp.dot(w_ref[...], feat_ref[...], preferred_element_type=jnp.float32)
    acc = acc + b_ref[...][:, None]
    o_ref[...] = _selu(acc)


def kernel(x, W0, b0, W1, b1, W2, b2, gamma2, beta2, W3, b3, W4, b4, Wf, bf):
    outs = []
    h = _edge_conv(x, W0, b0, _K); outs.append(h)
    h = _edge_conv(h, W1, b1, _K); outs.append(h)
    h = _edge_conv(h, W2, b2, _K, gamma2, beta2); outs.append(h)
    h = _edge_conv(h, W3, b3, _K); outs.append(h)
    h = _edge_conv(h, W4, b4, _K); outs.append(h)
    cat = jnp.concatenate(outs, axis=1)

    xs = cat[..., 0]
    B, C, N = xs.shape
    idx = _knn_idx(xs, 1)
    xt = jnp.transpose(xs, (0, 2, 1))
    xj = xt[jnp.arange(B)[:, None, None], idx]
    xj = jnp.transpose(xj[:, :, 0, :], (0, 2, 1))
    feat = jnp.concatenate([xs, xj - xs], axis=1)

    O = Wf.shape[0]
    out = pl.pallas_call(
        _final_kernel,
        grid=(B,),
        in_specs=[
            pl.BlockSpec((None, 2 * C, N), lambda b: (b, 0, 0)),
            pl.BlockSpec((O, 2 * C), lambda b: (0, 0)),
            pl.BlockSpec((O,), lambda b: (0,)),
        ],
        out_specs=pl.BlockSpec((None, O, N), lambda b: (b, 0, 0)),
        out_shape=jax.ShapeDtypeStruct((B, O, N), jnp.float32),
    )(feat, Wf, bf)
    return out[..., None]


# speedup vs baseline: 3.3656x; 3.3656x over previous
import functools

import jax
import jax.numpy as jnp
from jax.experimental import pallas as pl
from jax.experimental.pallas import tpu as pltpu

_K = 40
_NC = 128
_SELU_ALPHA = 1.6732632423543772
_SELU_SCALE = 1.0507009873554805


def _selu(x):
    return _SELU_SCALE * jnp.where(x > 0, x, _SELU_ALPHA * (jnp.exp(x) - 1.0))


def _nt_dot(a, b):
    return jax.lax.dot_general(
        a, b, (((1,), (1,)), ((), ())), preferred_element_type=jnp.float32,
        precision=jax.lax.Precision.HIGHEST)


def _dist_block(xs, xi):
    xi_t = jnp.transpose(xi)
    inner = -2.0 * jnp.dot(xi_t.astype(jnp.bfloat16), xs.astype(jnp.bfloat16),
                           preferred_element_type=jnp.float32)
    sqi = jnp.sum(xi_t * xi_t, axis=1, keepdims=True)
    sqj = jnp.sum(xs * xs, axis=0, keepdims=True)
    return (inner + sqi) + sqj


def _select_knn(d_ref, k):
    bn, n = d_ref.shape
    iota = jax.lax.broadcasted_iota(jnp.int32, (bn, n), 1)

    def body(t, carry):
        d = d_ref[...]
        m = jnp.min(d, axis=1, keepdims=True)
        first = jnp.min(jnp.where(d <= m, iota, n), axis=1, keepdims=True)
        d_ref[...] = jnp.where(iota == first, jnp.inf, d)
        return carry

    jax.lax.fori_loop(0, k, body, 0)
    return d_ref[...] == jnp.inf


def _edge_reduce(xs_ref, xi, w2b, d_ref, cout, want_stats):
    c, n = xs_ref.shape
    bn = xi.shape[1]
    nchunks = n // _NC

    def chunk_vals(t):
        j0 = t * _NC
        xsc = xs_ref[:, pl.ds(j0, _NC)]
        d3 = xsc[:, None, :] - xi[:, :, None]
        d2 = d3.astype(jnp.bfloat16).reshape(c, bn * _NC)
        p2 = jnp.dot(w2b, d2, preferred_element_type=jnp.float32)
        p2r = p2.reshape(cout, bn, _NC)
        selc = (d_ref[:, pl.ds(j0, _NC)] == jnp.inf)[None, :, :]
        return p2r, selc

    if not want_stats:
        def body(t, mx):
            p2r, selc = chunk_vals(t)
            hi = jnp.max(jnp.where(selc, p2r, -jnp.inf), axis=2)
            return jnp.maximum(mx, hi)

        mx = jax.lax.fori_loop(
            0, nchunks, body, jnp.full((cout, bn), -jnp.inf, jnp.float32))
        return mx, None, None, None, None

    def body(t, carry):
        mx, mn, s1, s2, cnt = carry
        p2r, selc = chunk_vals(t)
        hi = jnp.max(jnp.where(selc, p2r, -jnp.inf), axis=2)
        lo = jnp.min(jnp.where(selc, p2r, jnp.inf), axis=2)
        masked = jnp.where(selc, p2r, 0.0)
        ss = jnp.sum(masked, axis=2)
        qq = jnp.sum(masked * p2r, axis=2)
        cc = jnp.sum(selc[0].astype(jnp.float32), axis=1, keepdims=True)
        return (jnp.maximum(mx, hi), jnp.minimum(mn, lo), s1 + ss, s2 + qq,
                cnt + cc)

    init = (jnp.full((cout, bn), -jnp.inf, jnp.float32),
            jnp.full((cout, bn), jnp.inf, jnp.float32),
            jnp.zeros((cout, bn), jnp.float32),
            jnp.zeros((cout, bn), jnp.float32),
            jnp.zeros((bn, 1), jnp.float32))
    return jax.lax.fori_loop(0, nchunks, body, init)


def _edge_plain_kernel(xs_ref, xi_ref, w_ref, b_ref, h_ref, d_ref, *, cout, cin, k):
    xs = xs_ref[...]
    xi = xi_ref[...]
    d_ref[...] = _dist_block(xs, xi)
    _select_knn(d_ref, k)

    wb = w_ref[...].astype(jnp.bfloat16)
    w1b = wb[:, :cin]
    w2b = wb[:, cin:]
    a = jnp.dot(w1b, xi.astype(jnp.bfloat16), preferred_element_type=jnp.float32)
    a = a + jnp.transpose(b_ref[...])
    mx, _, _, _, _ = _edge_reduce(xs_ref, xi, w2b, d_ref, cout, False)
    h_ref[...] = _selu(a + mx)


def _edge_bn1_kernel(xs_ref, xi_ref, w_ref, b_ref,
                     a_ref, mx_ref, mn_ref, st_ref, d_ref, *, cout, cin, k):
    xs = xs_ref[...]
    xi = xi_ref[...]
    d_ref[...] = _dist_block(xs, xi)
    _select_knn(d_ref, k)

    wb = w_ref[...].astype(jnp.bfloat16)
    w1b = wb[:, :cin]
    w2b = wb[:, cin:]
    a = jnp.dot(w1b, xi.astype(jnp.bfloat16), preferred_element_type=jnp.float32)
    a = a + jnp.transpose(b_ref[...])
    a_ref[...] = a
    mx, mn, s1, s2, cnt = _edge_reduce(xs_ref, xi, w2b, d_ref, cout, True)
    mx_ref[...] = mx
    mn_ref[...] = mn

    cnt_row = jnp.transpose(cnt)
    sum1 = jnp.sum(cnt_row * a + s1, axis=1, keepdims=True)
    sum2 = jnp.sum(cnt_row * a * a + 2.0 * a * s1 + s2, axis=1, keepdims=True)
    r1 = jnp.transpose(sum1)
    r2 = jnp.transpose(sum2)
    z = jnp.zeros_like(r1)
    st_ref[...] = jnp.concatenate([r1, r2, z, z, z, z, z, z], axis=0)


def _bn2_kernel(a_ref, mx_ref, mn_ref, st_ref, g_ref, be_ref, h_ref, *, m_total):
    st = st_ref[...]
    ssum = jnp.sum(st, axis=0)
    mean = ssum[0:1, :] / m_total
    ex2 = ssum[1:2, :] / m_total
    var = ex2 - mean * mean
    g = g_ref[...] / jnp.sqrt(var + 1e-5)
    c = be_ref[...] - g * mean
    gcol = jnp.transpose(g)
    ccol = jnp.transpose(c)
    a = a_ref[...]
    e = jnp.where(gcol >= 0, mx_ref[...], mn_ref[...])
    h_ref[...] = _selu(gcol * (a + e) + ccol)


def _final_kernel(h0_ref, h1_ref, h2_ref, h3_ref, h4_ref,
                  x0_ref, x1_ref, x2_ref, x3_ref, x4_ref,
                  w_ref, b_ref, o_ref, *, cin):
    xs = jnp.concatenate(
        [h0_ref[...], h1_ref[...], h2_ref[...], h3_ref[...], h4_ref[...]], axis=0)
    xi = jnp.concatenate(
        [x0_ref[...], x1_ref[...], x2_ref[...], x3_ref[...], x4_ref[...]], axis=0)
    n = xs.shape[1]
    bn = xi.shape[1]
    d = _dist_block(xs, xi)
    m = jnp.min(d, axis=1, keepdims=True)
    iota = jax.lax.broadcasted_iota(jnp.int32, (bn, n), 1)
    t = jnp.min(jnp.where(d <= m, iota, n), axis=1, keepdims=True)
    onehot = jnp.where(iota == t, 1.0, 0.0)

    w = w_ref[...]
    w1 = w[:, :cin]
    w2 = w[:, cin:]
    p = jnp.dot(w2, xs, preferred_element_type=jnp.float32,
                precision=jax.lax.Precision.HIGHEST)
    psel = _nt_dot(onehot, p)
    a = jnp.dot(w1 - w2, xi, preferred_element_type=jnp.float32,
                precision=jax.lax.Precision.HIGHEST)
    a = a + jnp.transpose(b_ref[...])
    h = _selu(jnp.transpose(a) + psel)
    o_ref[...] = jnp.transpose(h)


def _row_block(n):
    return 256 if n % 256 == 0 else n


def _edge_layer(xs, w, b, k, gamma=None, beta=None):
    b_, c, n = xs.shape
    cout = w.shape[0]
    bn_blk = _row_block(n)
    nb = n // bn_blk
    b2 = b.reshape(1, cout)
    grid = (b_, nb)
    xs_spec = pl.BlockSpec((None, c, n), lambda bb, ii: (bb, 0, 0))
    xi_spec = pl.BlockSpec((None, c, bn_blk), lambda bb, ii: (bb, 0, ii))
    w_spec = pl.BlockSpec((cout, 2 * c), lambda bb, ii: (0, 0))
    bvec_spec = pl.BlockSpec((1, cout), lambda bb, ii: (0, 0))
    h_spec = pl.BlockSpec((None, cout, bn_blk), lambda bb, ii: (bb, 0, ii))
    scratch = [pltpu.VMEM((bn_blk, n), jnp.float32)]

    if gamma is None:
        kern = functools.partial(_edge_plain_kernel, cout=cout, cin=c, k=k)
        return pl.pallas_call(
            kern, grid=grid,
            in_specs=[xs_spec, xi_spec, w_spec, bvec_spec],
            out_specs=h_spec,
            out_shape=jax.ShapeDtypeStruct((b_, cout, n), jnp.float32),
            scratch_shapes=scratch,
        )(xs, xs, w, b2)

    g_total = b_ * nb
    kern1 = functools.partial(_edge_bn1_kernel, cout=cout, cin=c, k=k)
    st_spec = pl.BlockSpec((None, 8, cout), lambda bb, ii: (bb * nb + ii, 0, 0))
    a_arr, mx_arr, mn_arr, st_arr = pl.pallas_call(
        kern1, grid=grid,
        in_specs=[xs_spec, xi_spec, w_spec, bvec_spec],
        out_specs=[h_spec, h_spec, h_spec, st_spec],
        out_shape=[
            jax.ShapeDtypeStruct((b_, cout, n), jnp.float32),
            jax.ShapeDtypeStruct((b_, cout, n), jnp.float32),
            jax.ShapeDtypeStruct((b_, cout, n), jnp.float32),
            jax.ShapeDtypeStruct((g_total, 8, cout), jnp.float32),
        ],
        scratch_shapes=scratch,
    )(xs, xs, w, b2)

    m_total = float(b_ * n * k)
    kern2 = functools.partial(_bn2_kernel, m_total=m_total)
    stf_spec = pl.BlockSpec((g_total, 8, cout), lambda bb, ii: (0, 0, 0))
    return pl.pallas_call(
        kern2, grid=grid,
        in_specs=[h_spec, h_spec, h_spec, stf_spec, bvec_spec, bvec_spec],
        out_specs=h_spec,
        out_shape=jax.ShapeDtypeStruct((b_, cout, n), jnp.float32),
    )(a_arr, mx_arr, mn_arr, st_arr, gamma.reshape(1, cout), beta.reshape(1, cout))


def _final_layer(hs, w, b):
    b_, _, n = hs[0].shape
    cin = sum(h.shape[1] for h in hs)
    cout = w.shape[0]
    bn_blk = _row_block(n)
    nb = n // bn_blk
    grid = (b_, nb)
    kern = functools.partial(_final_kernel, cin=cin)
    full_specs = [pl.BlockSpec((None, h.shape[1], n), lambda bb, ii: (bb, 0, 0))
                  for h in hs]
    blk_specs = [pl.BlockSpec((None, h.shape[1], bn_blk), lambda bb, ii: (bb, 0, ii))
                 for h in hs]
    w_spec = pl.BlockSpec((cout, 2 * cin), lambda bb, ii: (0, 0))
    bvec_spec = pl.BlockSpec((1, cout), lambda bb, ii: (0, 0))
    o_spec = pl.BlockSpec((None, cout, bn_blk), lambda bb, ii: (bb, 0, ii))
    return pl.pallas_call(
        kern, grid=grid,
        in_specs=full_specs + blk_specs + [w_spec, bvec_spec],
        out_specs=o_spec,
        out_shape=jax.ShapeDtypeStruct((b_, cout, n), jnp.float32),
    )(*hs, *hs, w, b.reshape(1, cout))


def kernel(x, W0, b0, W1, b1, W2, b2, gamma2, beta2, W3, b3, W4, b4, Wf, bf):
    xs = x[..., 0]
    h0 = _edge_layer(xs, W0, b0, _K)
    h1 = _edge_layer(h0, W1, b1, _K)
    h2 = _edge_layer(h1, W2, b2, _K, gamma2, beta2)
    h3 = _edge_layer(h2, W3, b3, _K)
    h4 = _edge_layer(h3, W4, b4, _K)
    out = _final_layer([h0, h1, h2, h3, h4], Wf, bf)
    return out[..., None]
